# baseline (speedup 1.0000x reference)
import functools

import jax
import jax.numpy as jnp
from jax import lax
from jax.experimental import pallas as pl
from jax.experimental.pallas import tpu as pltpu
from jax.experimental.pallas import tpu_sc as plsc

D = 64
M = 128

_NC = 2
_NS = 16
_NW = _NC * _NS

_CHUNK = 512


def _gather_body(table_hbm, ids_hbm, out_hbm, idx_v, rows_v, sem):
    wid = lax.axis_index("s") * _NC + lax.axis_index("c")
    n = ids_hbm.shape[0]
    b_per_w = n // _NW
    n_chunks = b_per_w // _CHUNK
    base = wid * b_per_w

    def step(g, carry):
        off = base + g * _CHUNK
        pltpu.sync_copy(ids_hbm.at[pl.ds(off, _CHUNK)], idx_v)
        pltpu.async_copy(table_hbm.at[idx_v], rows_v, sem).wait()
        pltpu.sync_copy(rows_v, out_hbm.at[pl.ds(off, _CHUNK)])
        return carry

    lax.fori_loop(0, n_chunks, step, 0)


@functools.cache
def _make_gather(n):
    mesh = plsc.VectorSubcoreMesh(core_axis_name="c", subcore_axis_name="s")
    return pl.kernel(
        _gather_body,
        mesh=mesh,
        out_type=jax.ShapeDtypeStruct((n, 2 * D), jnp.float32),
        scratch_types=[
            pltpu.VMEM((_CHUNK,), jnp.int32),
            pltpu.VMEM((_CHUNK, 2 * D), jnp.float32),
            pltpu.SemaphoreType.DMA,
        ],
    )


def _proj_body(x_ref, ids_ref, w_ref, o_ref):
    blk = x_ref.shape[0]
    par = jnp.reshape(ids_ref[...], (blk,)) & 1
    col = lax.broadcasted_iota(jnp.int32, (blk, 2 * D), 1)
    x = jnp.where((col // D) == par[:, None], x_ref[...], 0.0)
    w2 = jnp.concatenate([w_ref[...], w_ref[...]], axis=1)
    o_ref[...] = lax.dot_general(
        x, w2,
        dimension_numbers=(((1,), (1,)), ((), ())),
        preferred_element_type=jnp.float32,
    )


def _project(x, ids3, w):
    n = x.shape[0]
    blk = 2048
    return pl.pallas_call(
        _proj_body,
        grid=(n // blk,),
        in_specs=[
            pl.BlockSpec((blk, 2 * D), lambda i: (i, 0)),
            pl.BlockSpec((1, 1, blk), lambda i: (i, 0, 0)),
            pl.BlockSpec((M, D), lambda i: (0, 0)),
        ],
        out_specs=pl.BlockSpec((blk, M), lambda i: (i, 0)),
        out_shape=jax.ShapeDtypeStruct((n, M), jnp.float32),
    )(x, ids3, w)


def kernel(input_ids, low_rank_embed, projection_w):
    bsz, seq = input_ids.shape
    ids = input_ids.reshape(-1).astype(jnp.int32)
    n = ids.shape[0]
    table2 = low_rank_embed.reshape(-1, 2 * D)
    pair_ids = ids >> 1
    rows2 = _make_gather(n)(table2, pair_ids)
    blk = 2048
    ids3 = ids.reshape(n // blk, 1, blk)
    out = _project(rows2, ids3, projection_w)
    return out.reshape(bsz, seq, M)


# speedup vs baseline: 1.0263x; 1.0263x over previous
import functools

import jax
import jax.numpy as jnp
from jax import lax
from jax.experimental import pallas as pl
from jax.experimental.pallas import tpu as pltpu
from jax.experimental.pallas import tpu_sc as plsc

D = 64
M = 128

_NC = 2
_NS = 16
_NW = _NC * _NS

_CHUNK = 512


def _gather_body(table_hbm, ids_hbm, out_hbm, idx_v, rows_v, sem):
    wid = lax.axis_index("s") * _NC + lax.axis_index("c")
    n = ids_hbm.shape[0]
    b_per_w = n // _NW
    n_chunks = b_per_w // _CHUNK
    base = wid * b_per_w

    def step(g, carry):
        off = base + g * _CHUNK
        pltpu.sync_copy(ids_hbm.at[pl.ds(off, _CHUNK)], idx_v)
        pltpu.async_copy(table_hbm.at[idx_v], rows_v, sem).wait()
        pltpu.sync_copy(rows_v, out_hbm.at[pl.ds(off, _CHUNK)])
        return carry

    lax.fori_loop(0, n_chunks, step, 0)


@functools.cache
def _make_gather(n):
    mesh = plsc.VectorSubcoreMesh(core_axis_name="c", subcore_axis_name="s")
    return pl.kernel(
        _gather_body,
        mesh=mesh,
        out_type=jax.ShapeDtypeStruct((n, D), jnp.float32),
        scratch_types=[
            pltpu.VMEM((_CHUNK,), jnp.int32),
            pltpu.VMEM((_CHUNK, D), jnp.float32),
            pltpu.SemaphoreType.DMA,
        ],
        compiler_params=pltpu.CompilerParams(use_tc_tiling_on_sc=False),
    )


def _proj_body(x_ref, w_ref, o_ref):
    o_ref[...] = lax.dot_general(
        x_ref[...], w_ref[...],
        dimension_numbers=(((1,), (1,)), ((), ())),
        preferred_element_type=jnp.float32,
    )


def _project(x, w):
    n = x.shape[0]
    blk = 16384
    return pl.pallas_call(
        _proj_body,
        grid=(n // blk,),
        in_specs=[
            pl.BlockSpec((blk, D), lambda i: (i, 0)),
            pl.BlockSpec((M, D), lambda i: (0, 0)),
        ],
        out_specs=pl.BlockSpec((blk, M), lambda i: (i, 0)),
        out_shape=jax.ShapeDtypeStruct((n, M), jnp.float32),
    )(x, w)


def kernel(input_ids, low_rank_embed, projection_w):
    bsz, seq = input_ids.shape
    ids = input_ids.reshape(-1).astype(jnp.int32)
    n = ids.shape[0]
    rows = _make_gather(n)(low_rank_embed, ids)
    out = _project(rows, projection_w)
    return out.reshape(bsz, seq, M)


# speedup vs baseline: 1.4023x; 1.3664x over previous
import functools

import jax
import jax.numpy as jnp
from jax import lax
from jax.experimental import pallas as pl
from jax.experimental.pallas import tpu as pltpu
from jax.experimental.pallas import tpu_sc as plsc

D = 64
M = 128

_NC = 2
_NS = 16
_NW = _NC * _NS

_CHUNK = 512
_TCBLK = 8192


def _gather_body(table_hbm, ids_hbm, out_hbm, idx_v, rows_v, sem):
    wid = lax.axis_index("s") * _NC + lax.axis_index("c")
    n = ids_hbm.shape[0]
    b_per_w = n // _NW
    n_chunks = b_per_w // _CHUNK
    base = wid * b_per_w

    def step(g, carry):
        off = base + g * _CHUNK
        pltpu.sync_copy(ids_hbm.at[pl.ds(off, _CHUNK)], idx_v)
        pltpu.async_copy(table_hbm.at[idx_v], rows_v, sem).wait()
        dst = (off // (2 * _TCBLK)) * _TCBLK + off % _TCBLK
        col = ((off // _TCBLK) % 2) * D
        pltpu.sync_copy(rows_v,
                        out_hbm.at[pl.ds(dst, _CHUNK), pl.ds(col, D)])
        return carry

    lax.fori_loop(0, n_chunks, step, 0)


@functools.cache
def _make_gather(n):
    mesh = plsc.VectorSubcoreMesh(core_axis_name="c", subcore_axis_name="s")
    return pl.kernel(
        _gather_body,
        mesh=mesh,
        out_type=jax.ShapeDtypeStruct((n // 2, 2 * D), jnp.float32),
        scratch_types=[
            pltpu.VMEM((_CHUNK,), jnp.int32),
            pltpu.VMEM((_CHUNK, D), jnp.float32),
            pltpu.SemaphoreType.DMA,
        ],
        compiler_params=pltpu.CompilerParams(use_tc_tiling_on_sc=False),
    )


def _proj_body(x_ref, w_ref, o_ref):
    blk = x_ref.shape[0]
    w = w_ref[...]
    dims = (((1,), (1,)), ((), ()))
    o_ref[0:blk, :] = lax.dot_general(
        x_ref[:, 0:D], w, dimension_numbers=dims,
        preferred_element_type=jnp.float32)
    o_ref[blk:2 * blk, :] = lax.dot_general(
        x_ref[:, D:2 * D], w, dimension_numbers=dims,
        preferred_element_type=jnp.float32)


def _project(x2, w):
    n2 = x2.shape[0]
    return pl.pallas_call(
        _proj_body,
        grid=(n2 // _TCBLK,),
        in_specs=[
            pl.BlockSpec((_TCBLK, 2 * D), lambda i: (i, 0)),
            pl.BlockSpec((M, D), lambda i: (0, 0)),
        ],
        out_specs=pl.BlockSpec((2 * _TCBLK, M), lambda i: (i, 0)),
        out_shape=jax.ShapeDtypeStruct((2 * n2, M), jnp.float32),
    )(x2, w)


def kernel(input_ids, low_rank_embed, projection_w):
    bsz, seq = input_ids.shape
    ids = input_ids.reshape(-1).astype(jnp.int32)
    n = ids.shape[0]
    rows2 = _make_gather(n)(low_rank_embed, ids)
    out = _project(rows2, projection_w)
    return out.reshape(bsz, seq, M)


# speedup vs baseline: 1.4094x; 1.0051x over previous
import functools

import jax
import jax.numpy as jnp
from jax import lax
from jax.experimental import pallas as pl
from jax.experimental.pallas import tpu as pltpu
from jax.experimental.pallas import tpu_sc as plsc

D = 64
M = 128

_NC = 2
_NS = 16
_NW = _NC * _NS

_CHUNK = 512
_TCBLK = 16384


def _gather_body(table_hbm, ids_hbm, out_hbm, idx_v, rows_v, sem):
    wid = lax.axis_index("s") * _NC + lax.axis_index("c")
    n = ids_hbm.shape[0]
    b_per_w = n // _NW
    n_chunks = b_per_w // _CHUNK
    base = wid * b_per_w

    def step(g, carry):
        off = base + g * _CHUNK
        pltpu.sync_copy(ids_hbm.at[pl.ds(off, _CHUNK)], idx_v)
        pltpu.async_copy(table_hbm.at[idx_v], rows_v, sem).wait()
        dst = (off // (2 * _TCBLK)) * _TCBLK + off % _TCBLK
        col = ((off // _TCBLK) % 2) * D
        pltpu.sync_copy(rows_v,
                        out_hbm.at[pl.ds(dst, _CHUNK), pl.ds(col, D)])
        return carry

    lax.fori_loop(0, n_chunks, step, 0)


@functools.cache
def _make_gather(n):
    mesh = plsc.VectorSubcoreMesh(core_axis_name="c", subcore_axis_name="s")
    return pl.kernel(
        _gather_body,
        mesh=mesh,
        out_type=jax.ShapeDtypeStruct((n // 2, 2 * D), jnp.float32),
        scratch_types=[
            pltpu.VMEM((_CHUNK,), jnp.int32),
            pltpu.VMEM((_CHUNK, D), jnp.float32),
            pltpu.SemaphoreType.DMA,
        ],
        compiler_params=pltpu.CompilerParams(use_tc_tiling_on_sc=False),
    )


def _proj_body(x_ref, w_ref, o_ref):
    blk = x_ref.shape[0]
    w = w_ref[...]
    dims = (((1,), (1,)), ((), ()))
    o_ref[0:blk, :] = lax.dot_general(
        x_ref[:, 0:D], w, dimension_numbers=dims,
        preferred_element_type=jnp.float32)
    o_ref[blk:2 * blk, :] = lax.dot_general(
        x_ref[:, D:2 * D], w, dimension_numbers=dims,
        preferred_element_type=jnp.float32)


def _project(x2, w):
    n2 = x2.shape[0]
    return pl.pallas_call(
        _proj_body,
        grid=(n2 // _TCBLK,),
        in_specs=[
            pl.BlockSpec((_TCBLK, 2 * D), lambda i: (i, 0)),
            pl.BlockSpec((M, D), lambda i: (0, 0)),
        ],
        out_specs=pl.BlockSpec((2 * _TCBLK, M), lambda i: (i, 0)),
        out_shape=jax.ShapeDtypeStruct((2 * n2, M), jnp.float32),
    )(x2, w)


def kernel(input_ids, low_rank_embed, projection_w):
    bsz, seq = input_ids.shape
    ids = input_ids.reshape(-1).astype(jnp.int32)
    n = ids.shape[0]
    rows2 = _make_gather(n)(low_rank_embed, ids)
    out = _project(rows2, projection_w)
    return out.reshape(bsz, seq, M)
